# baseline (speedup 1.0000x reference)
import functools

import jax
import jax.numpy as jnp
from jax import lax
from jax.experimental import pallas as pl
from jax.experimental.pallas import tpu as pltpu
from jax.experimental.pallas import tpu_sc as plsc

N_NODES = 10000
NP = 10240
N_EDGES = 320000
D = 128
NC, NS, L = 2, 16, 16
NW = NC * NS
RPT = NP // NW
ACC_R = RPT + 8
K = 120
W = 4000
NWIN = N_EDGES // W
SEL = W + 128
VEC16 = 16

_mesh = plsc.VectorSubcoreMesh(
    core_axis_name="c", subcore_axis_name="s", num_cores=NC, num_subcores=NS)


@functools.partial(
    pl.kernel,
    out_type=jax.ShapeDtypeStruct((NP, D), jnp.float32),
    mesh=_mesh,
    compiler_params=pltpu.CompilerParams(needs_layout_passes=False),
    scratch_types=[
        pltpu.VMEM((ACC_R, D), jnp.float32),
        pltpu.VMEM((W,), jnp.int32),
        pltpu.VMEM((W,), jnp.int32),
        pltpu.VMEM((SEL,), jnp.int32),
        pltpu.VMEM((SEL,), jnp.int32),
        pltpu.VMEM((K, D), jnp.float32),
        pltpu.SemaphoreType.DMA,
    ],
)
def _hop_kernel(y_hbm, src_hbm, dst_hbm, out_hbm,
                acc, src_w, dst_w, src_sel, dst_sel, rows_v, gsem):
    cid = lax.axis_index("c")
    sid = lax.axis_index("s")
    tid = cid * NS + sid
    lo = tid * RPT

    zf = jnp.zeros((VEC16,), jnp.float32)
    zi = jnp.zeros((VEC16,), jnp.int32)
    dumpv = jnp.full((VEC16,), RPT, jnp.int32)

    def zero_acc(r, carry):
        for c in range(D // VEC16):
            acc[r, pl.ds(c * VEC16, VEC16)] = zf
        return carry
    lax.fori_loop(0, ACC_R, zero_acc, 0)

    def window(w, carry):
        pltpu.sync_copy(src_hbm.at[pl.ds(w * W, W)], src_w)
        pltpu.sync_copy(dst_hbm.at[pl.ds(w * W, W)], dst_w)

        def scan(i, cur):
            dvec = dst_w[pl.ds(i * VEC16, VEC16)]
            svec = src_w[pl.ds(i * VEC16, VEC16)]
            lmask = jnp.logical_and(dvec >= lo, dvec < lo + RPT)
            mi = lmask.astype(jnp.int32)
            inc = plsc.cumsum(mi)
            pos = cur + (inc - mi)
            plsc.store_scatter(src_sel, [pos], svec, mask=lmask)
            plsc.store_scatter(dst_sel, [pos], dvec - lo, mask=lmask)
            return cur + jnp.max(inc)
        cur = lax.fori_loop(0, W // VEC16, scan, jnp.int32(0))

        iota16 = lax.iota(jnp.int32, VEC16)
        for t in range(8):
            ppos = cur + (t * VEC16) + iota16
            plsc.store_scatter(src_sel, [ppos], zi)
            plsc.store_scatter(dst_sel, [ppos], dumpv)

        nch = (cur + (K - 1)) // K

        def chunk(g, carry):
            base = g * K
            pltpu.async_copy(
                y_hbm.at[src_sel.at[pl.ds(base, K)]], rows_v, gsem).wait()

            def row(j, c2):
                dlv = plsc.load_gather(
                    dst_sel, [jnp.full((VEC16,), base + j, jnp.int32)])
                dl = jnp.max(dlv)
                for c in range(D // VEC16):
                    plsc.addupdate(acc.at[dl, pl.ds(c * VEC16, VEC16)],
                                   rows_v[j, pl.ds(c * VEC16, VEC16)])
                return c2
            lax.fori_loop(0, K, row, 0)
            return carry
        lax.fori_loop(0, nch, chunk, 0)
        return carry
    lax.fori_loop(0, NWIN, window, 0)

    pltpu.sync_copy(acc.at[pl.ds(0, RPT)], out_hbm.at[pl.ds(lo, RPT)])


def _norm_body(deg_ref, x_ref, y_ref, n1_ref, n2_ref):
    deg = jnp.maximum(deg_ref[:N_NODES], 1.0)
    n1 = lax.rsqrt(deg)
    n1_ref[...] = n1
    n2_ref[...] = 1.0 / deg
    y_ref[...] = x_ref[...] * n1


_norm_scale = pl.pallas_call(
    _norm_body,
    out_shape=(
        jax.ShapeDtypeStruct((N_NODES, D), jnp.float32),
        jax.ShapeDtypeStruct((N_NODES, D), jnp.float32),
        jax.ShapeDtypeStruct((N_NODES, D), jnp.float32),
    ),
)


def _mid_body(z_ref, n2_ref, y2_ref):
    y2_ref[...] = z_ref[:N_NODES] * n2_ref[...]


_mid_scale = pl.pallas_call(
    _mid_body,
    out_shape=jax.ShapeDtypeStruct((N_NODES, D), jnp.float32),
)


def _final_body(u_ref, n1_ref, w_ref, b_ref, o_ref):
    s = u_ref[:N_NODES] * n1_ref[...]
    o_ref[...] = lax.dot_general(
        s, w_ref[...], (((1,), (1,)), ((), ())),
        preferred_element_type=jnp.float32) + b_ref[...]


_final = pl.pallas_call(
    _final_body,
    out_shape=jax.ShapeDtypeStruct((N_NODES, D), jnp.float32),
)


def kernel(x, edge_index, W_mat, b):
    src = edge_index[0].astype(jnp.int32)
    dst = edge_index[1].astype(jnp.int32)
    ones = jnp.ones((N_NODES, D), jnp.float32)
    deg128 = _hop_kernel(ones, src, dst)
    y, n1, n2 = _norm_scale(deg128, x)
    z = _hop_kernel(y, src, dst)
    y2 = _mid_scale(z, n2)
    u = _hop_kernel(y2, src, dst)
    return _final(u, n1, W_mat, b.reshape(1, D))


# speedup vs baseline: 2.7245x; 2.7245x over previous
import functools

import jax
import jax.numpy as jnp
from jax import lax
from jax.experimental import pallas as pl
from jax.experimental.pallas import tpu as pltpu
from jax.experimental.pallas import tpu_sc as plsc

N_NODES = 10000
NP = 10240
N_EDGES = 320000
D = 128
NC, NS = 2, 16
NW = NC * NS
RPT = NP // NW
ACC_R = RPT + 8
K = 120
W = 8000
NWIN = N_EDGES // W
SEL = W + 128
V = 16

_mesh = plsc.VectorSubcoreMesh(
    core_axis_name="c", subcore_axis_name="s", num_cores=NC, num_subcores=NS)


def _make_pass(gather):
    fw = D if gather else V

    @functools.partial(
        pl.kernel,
        out_type=jax.ShapeDtypeStruct((NP * fw,), jnp.float32),
        mesh=_mesh,
        compiler_params=pltpu.CompilerParams(needs_layout_passes=False),
        scratch_types=[
            pltpu.VMEM((ACC_R * fw,), jnp.float32),
            pltpu.VMEM((W,), jnp.int32),
            pltpu.VMEM((W,), jnp.int32),
            pltpu.VMEM((SEL,), jnp.int32),
            pltpu.VMEM((SEL,), jnp.int32),
            pltpu.VMEM((K, D), jnp.float32),
            pltpu.SemaphoreType.DMA,
        ],
    )
    def _pass(y_hbm, src_hbm, dst_hbm, out_hbm,
              acc, src_w, dst_w, src_sel, dst_sel, rows_v, gsem):
        cid = lax.axis_index("c")
        sid = lax.axis_index("s")
        tid = cid * NS + sid
        lo = tid * RPT

        zf = jnp.zeros((V,), jnp.float32)
        zi = jnp.zeros((V,), jnp.int32)
        onev = jnp.ones((V,), jnp.int32)
        onesf = jnp.ones((V,), jnp.float32)
        iota16 = lax.iota(jnp.int32, V)

        def zero_acc(r, carry):
            acc[pl.ds(r * V, V)] = zf
            return carry
        lax.fori_loop(0, ACC_R * fw // V, zero_acc, 0)

        def window(w, carry):
            pltpu.sync_copy(src_hbm.at[pl.ds(w * W, W)], src_w)
            pltpu.sync_copy(dst_hbm.at[pl.ds(w * W, W)], dst_w)

            def scan(i, cur_v):
                dvec = dst_w[pl.ds(i * V, V)]
                svec = src_w[pl.ds(i * V, V)]
                t = dvec - lo
                lmask = (t | ((RPT - 1) - t)) >= 0
                mi = lmask.astype(jnp.int32)
                inc = plsc.cumsum(mi)
                pos = cur_v + (inc - mi)
                plsc.store_scatter(src_sel, [pos], svec, mask=lmask)
                plsc.store_scatter(dst_sel, [pos], t, mask=lmask)
                return cur_v + plsc.all_reduce_population_count(lmask)
            cur_v = lax.fori_loop(0, W // V, scan, jnp.zeros((V,), jnp.int32))
            cur = jnp.max(cur_v)

            for p in range(8):
                plsc.store_scatter(src_sel, [cur_v + (p * V) + iota16], zi)

            nch = (cur + (K - 1)) // K

            def chunk(g, bv):
                base = g * K
                if gather:
                    pltpu.async_copy(
                        y_hbm.at[src_sel.at[pl.ds(base, K)]], rows_v,
                        gsem).wait()
                rlim = jnp.minimum(jnp.int32(K), cur - base)

                def row(j, bv):
                    dlv = plsc.load_gather(dst_sel, [bv])
                    addr = dlv * fw
                    if gather:
                        for c in range(D // V):
                            plsc.addupdate_scatter(
                                acc, [addr + (iota16 + c * V)],
                                rows_v[j, pl.ds(c * V, V)])
                    else:
                        plsc.addupdate_scatter(acc, [addr + iota16], onesf)
                    return bv + onev
                return lax.fori_loop(0, rlim, row, bv)
            lax.fori_loop(0, nch, chunk, jnp.zeros((V,), jnp.int32))
            return carry
        lax.fori_loop(0, NWIN, window, 0)

        pltpu.sync_copy(acc.at[pl.ds(0, RPT * fw)],
                        out_hbm.at[pl.ds(tid * RPT * fw, RPT * fw)])

    return _pass


_deg_pass = _make_pass(gather=False)
_hop_pass = _make_pass(gather=True)


def _norm_body(deg_ref, x_ref, y_ref, n1_ref, n2_ref):
    deg = jnp.maximum(deg_ref[:N_NODES, :1], 1.0)
    n1 = lax.rsqrt(deg)
    n1_ref[...] = jnp.broadcast_to(n1, (N_NODES, D))
    n2_ref[...] = jnp.broadcast_to(1.0 / deg, (N_NODES, D))
    y_ref[...] = x_ref[...] * n1


_norm_scale = pl.pallas_call(
    _norm_body,
    out_shape=(
        jax.ShapeDtypeStruct((N_NODES, D), jnp.float32),
        jax.ShapeDtypeStruct((N_NODES, D), jnp.float32),
        jax.ShapeDtypeStruct((N_NODES, D), jnp.float32),
    ),
)


def _mid_body(z_ref, n2_ref, y2_ref):
    y2_ref[...] = z_ref[:N_NODES] * n2_ref[...]


_mid_scale = pl.pallas_call(
    _mid_body,
    out_shape=jax.ShapeDtypeStruct((N_NODES, D), jnp.float32),
)


def _final_body(u_ref, n1_ref, w_ref, b_ref, o_ref):
    s = u_ref[:N_NODES] * n1_ref[...]
    o_ref[...] = lax.dot_general(
        s, w_ref[...], (((1,), (1,)), ((), ())),
        preferred_element_type=jnp.float32) + b_ref[...]


_final = pl.pallas_call(
    _final_body,
    out_shape=jax.ShapeDtypeStruct((N_NODES, D), jnp.float32),
)


def kernel(x, edge_index, W_mat, b):
    src = edge_index[0].astype(jnp.int32)
    dst = edge_index[1].astype(jnp.int32)
    deg16 = _deg_pass(x, src, dst).reshape(NP, V)
    y, n1, n2 = _norm_scale(deg16, x)
    z = _hop_pass(y, src, dst).reshape(NP, D)
    y2 = _mid_scale(z, n2)
    u = _hop_pass(y2, src, dst).reshape(NP, D)
    return _final(u, n1, W_mat, b.reshape(1, D))


# speedup vs baseline: 11.5994x; 4.2575x over previous
import functools

import jax
import jax.numpy as jnp
from jax import lax
from jax.experimental import pallas as pl
from jax.experimental.pallas import tpu as pltpu
from jax.experimental.pallas import tpu_sc as plsc

N_NODES = 10000
NP = 10240
N_EDGES = 320000
D = 128
NC, NS = 2, 16
NW = NC * NS
RPT = NP // NW
ACC_R = RPT + 8
K = 40
NBUF = 8
W = 4000
NWIN = N_EDGES // W
CAP = 14000
THRESH = 9600
V = 16

_mesh = plsc.VectorSubcoreMesh(
    core_axis_name="c", subcore_axis_name="s", num_cores=NC, num_subcores=NS)


def _make_pass(gather):
    fw = D if gather else V

    @functools.partial(
        pl.kernel,
        out_type=jax.ShapeDtypeStruct((NP * fw,), jnp.float32),
        mesh=_mesh,
        compiler_params=pltpu.CompilerParams(needs_layout_passes=False),
        scratch_types=[
            pltpu.VMEM((ACC_R * fw,), jnp.float32),
            pltpu.VMEM((W,), jnp.int32),
            pltpu.VMEM((W,), jnp.int32),
            pltpu.VMEM((CAP,), jnp.int32),
            pltpu.VMEM((CAP,), jnp.int32),
            pltpu.VMEM((NBUF, K, D), jnp.float32),
        ] + [pltpu.SemaphoreType.DMA] * NBUF,
    )
    def _pass(y_hbm, src_hbm, dst_hbm, out_hbm,
              acc, src_w, dst_w, src_sel, dst_sel, rows_v, *gsems):
        cid = lax.axis_index("c")
        sid = lax.axis_index("s")
        tid = cid * NS + sid
        lo = tid * RPT

        zf = jnp.zeros((V,), jnp.float32)
        zi = jnp.zeros((V,), jnp.int32)
        onev = jnp.ones((V,), jnp.int32)
        onesf = jnp.ones((V,), jnp.float32)
        iota16 = lax.iota(jnp.int32, V)

        def zero_acc(r, carry):
            acc[pl.ds(r * V, V)] = zf
            return carry
        lax.fori_loop(0, ACC_R * fw // V, zero_acc, 0)

        def rows_of_chunk(g, cur, b):
            base = g * K
            rlim = jnp.minimum(jnp.int32(K), cur - base)

            def row(j, carry):
                bv = jnp.full((V,), base + j, jnp.int32)
                dlv = plsc.load_gather(dst_sel, [bv])
                addr = dlv * fw
                if gather:
                    for c in range(D // V):
                        plsc.addupdate_scatter(
                            acc, [addr + (iota16 + c * V)],
                            rows_v[b, j, pl.ds(c * V, V)])
                else:
                    plsc.addupdate_scatter(acc, [addr + iota16], onesf)
                return carry
            lax.fori_loop(0, rlim, row, 0)

        def fire(g, b):
            pltpu.make_async_copy(
                y_hbm.at[src_sel.at[pl.ds(g * K, K)]], rows_v.at[b],
                gsems[b]).start()

        def wait(g, b):
            pltpu.make_async_copy(
                y_hbm.at[src_sel.at[pl.ds(g * K, K)]], rows_v.at[b],
                gsems[b]).wait()

        def drain(cur_v):
            cur = jnp.max(cur_v)
            for p in range(8):
                plsc.store_scatter(src_sel, [cur_v + (p * V) + iota16], zi)
            nch = (cur + (K - 1)) // K
            if gather:
                for b in range(NBUF):
                    @pl.when(b < nch)
                    def _():
                        fire(b, b)

                def rnd(t, carry):
                    for b in range(NBUF):
                        g = t * NBUF + b

                        @pl.when(g < nch)
                        def _():
                            wait(g, b)
                            rows_of_chunk(g, cur, b)

                            @pl.when(g + NBUF < nch)
                            def _():
                                fire(g + NBUF, b)
                    return carry
                lax.fori_loop(0, (nch + NBUF - 1) // NBUF, rnd, 0)
            else:
                def chunk(g, carry):
                    rows_of_chunk(g, cur, 0)
                    return carry
                lax.fori_loop(0, nch, chunk, 0)

        def window(w, cur_v):
            pltpu.sync_copy(src_hbm.at[pl.ds(w * W, W)], src_w)
            pltpu.sync_copy(dst_hbm.at[pl.ds(w * W, W)], dst_w)

            def scan(i, cur_v):
                dvec = dst_w[pl.ds(i * V, V)]
                svec = src_w[pl.ds(i * V, V)]
                t = dvec - lo
                lmask = (t | ((RPT - 1) - t)) >= 0
                mi = lmask.astype(jnp.int32)
                inc = plsc.cumsum(mi)
                pos = cur_v + (inc - mi)
                plsc.store_scatter(src_sel, [pos], svec, mask=lmask)
                plsc.store_scatter(dst_sel, [pos], t, mask=lmask)
                return cur_v + plsc.all_reduce_population_count(lmask)
            cur_v = lax.fori_loop(0, W // V, scan, cur_v)

            full = jnp.max(cur_v) >= THRESH

            @pl.when(full)
            def _():
                drain(cur_v)
            return jnp.where(full, jnp.zeros((V,), jnp.int32), cur_v)
        cur_v = lax.fori_loop(0, NWIN, window, jnp.zeros((V,), jnp.int32))

        @pl.when(jnp.max(cur_v) > 0)
        def _():
            drain(cur_v)

        pltpu.sync_copy(acc.at[pl.ds(0, RPT * fw)],
                        out_hbm.at[pl.ds(tid * RPT * fw, RPT * fw)])

    return _pass


_deg_pass = _make_pass(gather=False)
_hop_pass = _make_pass(gather=True)


def _norm_body(deg_ref, x_ref, y_ref, n1_ref, n2_ref):
    deg = jnp.maximum(deg_ref[:N_NODES, :1], 1.0)
    n1 = lax.rsqrt(deg)
    n1_ref[...] = jnp.broadcast_to(n1, (N_NODES, D))
    n2_ref[...] = jnp.broadcast_to(1.0 / deg, (N_NODES, D))
    y_ref[...] = x_ref[...] * n1


_norm_scale = pl.pallas_call(
    _norm_body,
    out_shape=(
        jax.ShapeDtypeStruct((N_NODES, D), jnp.float32),
        jax.ShapeDtypeStruct((N_NODES, D), jnp.float32),
        jax.ShapeDtypeStruct((N_NODES, D), jnp.float32),
    ),
)


def _mid_body(z_ref, n2_ref, y2_ref):
    y2_ref[...] = z_ref[:N_NODES] * n2_ref[...]


_mid_scale = pl.pallas_call(
    _mid_body,
    out_shape=jax.ShapeDtypeStruct((N_NODES, D), jnp.float32),
)


def _final_body(u_ref, n1_ref, w_ref, b_ref, o_ref):
    s = u_ref[:N_NODES] * n1_ref[...]
    o_ref[...] = lax.dot_general(
        s, w_ref[...], (((1,), (1,)), ((), ())),
        preferred_element_type=jnp.float32) + b_ref[...]


_final = pl.pallas_call(
    _final_body,
    out_shape=jax.ShapeDtypeStruct((N_NODES, D), jnp.float32),
)


def kernel(x, edge_index, W_mat, b):
    src = edge_index[0].astype(jnp.int32)
    dst = edge_index[1].astype(jnp.int32)
    deg16 = _deg_pass(x, src, dst).reshape(NP, V)
    y, n1, n2 = _norm_scale(deg16, x)
    z = _hop_pass(y, src, dst).reshape(NP, D)
    y2 = _mid_scale(z, n2)
    u = _hop_pass(y2, src, dst).reshape(NP, D)
    return _final(u, n1, W_mat, b.reshape(1, D))
